# baseline (speedup 1.0000x reference)
import functools
import math

import jax
import jax.numpy as jnp
from jax import lax
from jax.experimental import pallas as pl
from jax.experimental.pallas import tpu as pltpu
from jax.experimental.pallas import tpu_sc as plsc

_NUM_CORES = 2
_NUM_SUBCORES = 16
_LANES = 16
_NUM_WORKERS = _NUM_CORES * _NUM_SUBCORES

_NSET = 3
_LOOKAHEAD = 2
_JUNROLL = 4


@functools.partial(jax.jit, static_argnames=("batch", "seq", "d"))
def _embed_sc(x_flat, table, pe, *, batch, seq, d):
    s_per_w = seq // _NUM_WORKERS
    chunk = 8
    n_steps = s_per_w // chunk
    scale = float(math.sqrt(d))
    groups = d // _LANES

    mesh = plsc.VectorSubcoreMesh(core_axis_name="c", subcore_axis_name="s")

    @functools.partial(
        pl.kernel,
        out_type=jax.ShapeDtypeStruct((batch * seq, d), jnp.float32),
        mesh=mesh,
        scratch_types=[
            pltpu.VMEM((batch * s_per_w,), jnp.int32),
            [[pltpu.VMEM((chunk, d), jnp.float32)] * batch] * _NSET,
            [pltpu.VMEM((chunk, d), jnp.float32)] * 2,
            [[pltpu.SemaphoreType.DMA] * batch] * _NSET,
            [[pltpu.SemaphoreType.DMA] * batch] * _NSET,
            [pltpu.SemaphoreType.DMA] * 2,
        ],
    )
    def k(x_hbm, table_hbm, pe_hbm, out_hbm, idx_v, sets, pes, gsem, ssem,
          psem):
        wid = lax.axis_index("s") * _NUM_CORES + lax.axis_index("c")
        s0 = wid * s_per_w
        for b in range(batch):
            pltpu.sync_copy(
                x_hbm.at[pl.ds(b * seq + s0, s_per_w)],
                idx_v.at[pl.ds(b * s_per_w, s_per_w)],
            )

        def gather(t, b):
            return pltpu.async_copy(
                table_hbm.at[idx_v.at[pl.ds(b * s_per_w + t * chunk, chunk)]],
                sets[t % _NSET][b],
                gsem[t % _NSET][b],
            )

        def store(t, b):
            return pltpu.async_copy(
                sets[t % _NSET][b],
                out_hbm.at[pl.ds(b * seq + s0 + t * chunk, chunk)],
                ssem[t % _NSET][b],
            )

        def load_pe(t):
            return pltpu.async_copy(
                pe_hbm.at[pl.ds(s0 + t * chunk, chunk)],
                pes[t % 2],
                psem[t % 2],
            )

        gathers = [[None] * batch for _ in range(_NSET)]
        stores = [[None] * batch for _ in range(_NSET)]
        pe_loads = [load_pe(0), load_pe(1)]

        for t in range(_LOOKAHEAD):
            for b in range(batch):
                gathers[t % _NSET][b] = gather(t, b)

        for t in range(n_steps):
            st = t % _NSET
            ta = t + _LOOKAHEAD
            if ta < n_steps:
                sa = ta % _NSET
                for b in range(batch):
                    if stores[sa][b] is not None:
                        stores[sa][b].wait()
                    gathers[sa][b] = gather(ta, b)
            pe_loads[t % 2].wait()
            pv = pes[t % 2]
            for b in range(batch):
                gathers[st][b].wait()
            bufs = sets[st]

            def row_body(r, _):
                def grp_body(j0, _):
                    for u in range(_JUNROLL):
                        sl = pl.ds((j0 * _JUNROLL + u) * _LANES, _LANES)
                        p = pv[r, sl]
                        for b in range(batch):
                            bufs[b][r, sl] = bufs[b][r, sl] * scale + p
                    return 0

                return lax.fori_loop(0, groups // _JUNROLL, grp_body, 0)

            lax.fori_loop(0, chunk, row_body, 0)
            for b in range(batch):
                stores[st][b] = store(t, b)
            if t + 2 < n_steps:
                pe_loads[t % 2] = load_pe(t + 2)

        for row in stores:
            for s in row:
                if s is not None:
                    s.wait()

    return k(x_flat, table, pe)


def kernel(x, table, pe):
    batch, seq = x.shape
    d = table.shape[1]
    x_flat = x.reshape(-1).astype(jnp.int32)
    out = _embed_sc(x_flat, table, pe, batch=batch, seq=seq, d=d)
    return out.reshape(batch, seq, d)


# speedup vs baseline: 1.4853x; 1.4853x over previous
import functools
import math

import jax
import jax.numpy as jnp
from jax import lax
from jax.experimental import pallas as pl
from jax.experimental.pallas import tpu as pltpu
from jax.experimental.pallas import tpu_sc as plsc

_NUM_CORES = 2
_NUM_SUBCORES = 16
_LANES = 16
_NUM_WORKERS = _NUM_CORES * _NUM_SUBCORES

_NSET = 3
_LOOKAHEAD = 2
_DSPLIT = 2
_JUNROLL = 4


@functools.partial(jax.jit, static_argnames=("batch", "seq", "d"))
def _embed_sc(x_flat, table, pe, *, batch, seq, d):
    s_per_w = seq // _NUM_WORKERS
    chunk = 16
    n_chunks = s_per_w // chunk
    dh = d // _DSPLIT
    scale = float(math.sqrt(d))
    groups = dh // _LANES
    n_steps = n_chunks * _DSPLIT

    mesh = plsc.VectorSubcoreMesh(core_axis_name="c", subcore_axis_name="s")

    @functools.partial(
        pl.kernel,
        out_type=jax.ShapeDtypeStruct((batch * seq, d), jnp.float32),
        mesh=mesh,
        scratch_types=[
            pltpu.VMEM((batch * s_per_w,), jnp.int32),
            [[pltpu.VMEM((chunk, dh), jnp.float32)] * batch] * _NSET,
            [pltpu.VMEM((chunk, dh), jnp.float32)] * 2,
            [[pltpu.SemaphoreType.DMA] * batch] * _NSET,
            [[pltpu.SemaphoreType.DMA] * batch] * _NSET,
            [pltpu.SemaphoreType.DMA] * 2,
        ],
    )
    def k(x_hbm, table_hbm, pe_hbm, out_hbm, idx_v, sets, pes, gsem, ssem,
          psem):
        wid = lax.axis_index("s") * _NUM_CORES + lax.axis_index("c")
        s0 = wid * s_per_w
        for b in range(batch):
            pltpu.sync_copy(
                x_hbm.at[pl.ds(b * seq + s0, s_per_w)],
                idx_v.at[pl.ds(b * s_per_w, s_per_w)],
            )

        def gather(t, b):
            c, h = divmod(t, _DSPLIT)
            return pltpu.async_copy(
                table_hbm.at[
                    idx_v.at[pl.ds(b * s_per_w + c * chunk, chunk)],
                    pl.ds(h * dh, dh),
                ],
                sets[t % _NSET][b],
                gsem[t % _NSET][b],
            )

        def store(t, b):
            c, h = divmod(t, _DSPLIT)
            return pltpu.async_copy(
                sets[t % _NSET][b],
                out_hbm.at[
                    pl.ds(b * seq + s0 + c * chunk, chunk),
                    pl.ds(h * dh, dh),
                ],
                ssem[t % _NSET][b],
            )

        def load_pe(t):
            c, h = divmod(t, _DSPLIT)
            return pltpu.async_copy(
                pe_hbm.at[pl.ds(s0 + c * chunk, chunk), pl.ds(h * dh, dh)],
                pes[t % 2],
                psem[t % 2],
            )

        gathers = [[None] * batch for _ in range(_NSET)]
        stores = [[None] * batch for _ in range(_NSET)]
        pe_loads = [load_pe(0), load_pe(1)]

        for t in range(_LOOKAHEAD):
            for b in range(batch):
                gathers[t % _NSET][b] = gather(t, b)

        for t in range(n_steps):
            st = t % _NSET
            ta = t + _LOOKAHEAD
            if ta < n_steps:
                sa = ta % _NSET
                for b in range(batch):
                    if stores[sa][b] is not None:
                        stores[sa][b].wait()
                    gathers[sa][b] = gather(ta, b)
            pe_loads[t % 2].wait()
            pv = pes[t % 2]
            for b in range(batch):
                gathers[st][b].wait()
            bufs = sets[st]

            def row_body(r, _):
                def grp_body(j0, _):
                    for u in range(_JUNROLL):
                        sl = pl.ds((j0 * _JUNROLL + u) * _LANES, _LANES)
                        p = pv[r, sl]
                        for b in range(batch):
                            bufs[b][r, sl] = bufs[b][r, sl] * scale + p
                    return 0

                return lax.fori_loop(0, groups // _JUNROLL, grp_body, 0)

            lax.fori_loop(0, chunk, row_body, 0)
            for b in range(batch):
                stores[st][b] = store(t, b)
            if t + 2 < n_steps:
                pe_loads[t % 2] = load_pe(t + 2)

        for row in stores:
            for s in row:
                if s is not None:
                    s.wait()

    return k(x_flat, table, pe)


def kernel(x, table, pe):
    batch, seq = x.shape
    d = table.shape[1]
    x_flat = x.reshape(-1).astype(jnp.int32)
    out = _embed_sc(x_flat, table, pe, batch=batch, seq=seq, d=d)
    return out.reshape(batch, seq, d)
